# baseline (speedup 1.0000x reference)
import functools

import jax
import jax.numpy as jnp
from jax import lax
from jax.experimental import pallas as pl
from jax.experimental.pallas import tpu as pltpu
from jax.experimental.pallas import tpu_sc as plsc

N_USERS = 1000000
N_ITEMS = 1000000
D = 16
B = 16384

NC = 2
NS = 16
L = 16
NW = NC * NS
BPW = B // NW
NBLK = BPW // L


def _sc_body(x_hbm, emb_hbm, bias_hbm, out_hbm,
             x_v, uidx, iidx, urows, irows, ub, ib, outv, sem):
    wid = lax.axis_index("s") * NC + lax.axis_index("c")
    base = wid * BPW

    pltpu.sync_copy(x_hbm.at[pl.ds(base, BPW), :], x_v)

    lane = lax.iota(jnp.int32, L)
    zeros = jnp.zeros((L,), jnp.int32)
    ones = jnp.full((L,), 1, jnp.int32)

    def build(blk, _):
        row = blk * L + lane
        u = plsc.load_gather(x_v, [row, zeros])
        it = plsc.load_gather(x_v, [row, ones]) + N_USERS
        uidx[pl.ds(blk * L, L)] = u
        iidx[pl.ds(blk * L, L)] = it
        return 0

    lax.fori_loop(0, NBLK, build, 0)

    cu = pltpu.async_copy(emb_hbm.at[uidx], urows, sem)
    ci = pltpu.async_copy(emb_hbm.at[iidx], irows, sem)
    cbu = pltpu.async_copy(bias_hbm.at[uidx], ub, sem)
    cbi = pltpu.async_copy(bias_hbm.at[iidx], ib, sem)
    cu.wait()
    ci.wait()
    cbu.wait()
    cbi.wait()

    def dot_blk(blk, _):
        row = blk * L + lane
        acc = ub[pl.ds(blk * L, L)] + ib[pl.ds(blk * L, L)]
        for d in range(D):
            dvec = jnp.full((L,), d, jnp.int32)
            uc = plsc.load_gather(urows, [row, dvec])
            ic = plsc.load_gather(irows, [row, dvec])
            acc = acc + uc * ic
        outv[pl.ds(blk * L, L)] = acc
        return 0

    lax.fori_loop(0, NBLK, dot_blk, 0)

    pltpu.sync_copy(outv, out_hbm.at[pl.ds(base, BPW)])


@jax.jit
def _gmf(x_batch, emb_table, bias_flat):
    mesh = plsc.VectorSubcoreMesh(
        core_axis_name="c", subcore_axis_name="s",
        num_cores=NC, num_subcores=NS)
    return pl.kernel(
        _sc_body,
        out_type=jax.ShapeDtypeStruct((B,), jnp.float32),
        mesh=mesh,
        compiler_params=pltpu.CompilerParams(
            needs_layout_passes=False, use_tc_tiling_on_sc=False),
        scratch_types=[
            pltpu.VMEM((BPW, 2), jnp.int32),
            pltpu.VMEM((BPW,), jnp.int32),
            pltpu.VMEM((BPW,), jnp.int32),
            pltpu.VMEM((BPW, D), jnp.float32),
            pltpu.VMEM((BPW, D), jnp.float32),
            pltpu.VMEM((BPW,), jnp.float32),
            pltpu.VMEM((BPW,), jnp.float32),
            pltpu.VMEM((BPW,), jnp.float32),
            pltpu.SemaphoreType.DMA,
        ],
    )(x_batch, emb_table, bias_flat)


def kernel(x_batch, emb_table, bias_table):
    x = x_batch.astype(jnp.int32)
    bias_flat = bias_table.reshape(-1)
    return _gmf(x, emb_table, bias_flat)


# speedup vs baseline: 7.6331x; 7.6331x over previous
import jax
import jax.numpy as jnp
from jax import lax
from jax.experimental import pallas as pl
from jax.experimental.pallas import tpu as pltpu
from jax.experimental.pallas import tpu_sc as plsc

N_USERS = 1000000
N_ITEMS = 1000000
D = 16
B = 16384

NC = 2
NS = 16
L = 16
NW = NC * NS
BPW = B // NW
NBLK = BPW // L

ROWS = N_USERS + N_ITEMS
RG = ROWS // 128
FG_STRIDE = RG * 1024

DOFF = [(d // 8) * FG_STRIDE + (d % 8) * 128 for d in range(D)]


def _sc_body(x_hbm, emb_hbm, bias_hbm, out_hbm,
             xv, uix, iix, ruix, riix, uft, ift, ub, ib, outv, sem):
    wid = lax.axis_index("s") * NC + lax.axis_index("c")
    base = wid * BPW

    pltpu.sync_copy(x_hbm.at[pl.ds(base * 2, 2 * BPW)], xv)

    def build(blk, _):
        qoff = (blk // 8) * 256 + (blk % 8) * L
        ru = xv[pl.ds(qoff, L)]
        ri = xv[pl.ds(qoff + 128, L)] + N_USERS
        ruix[pl.ds(blk * L, L)] = ru
        riix[pl.ds(blk * L, L)] = ri
        bu = ((ru >> 7) << 10) + (ru & 127)
        bi = ((ri >> 7) << 10) + (ri & 127)
        for d in range(D):
            uix[pl.ds(d * BPW + blk * L, L)] = bu + DOFF[d]
            iix[pl.ds(d * BPW + blk * L, L)] = bi + DOFF[d]
        return 0

    lax.fori_loop(0, NBLK, build, 0)

    cu = pltpu.async_copy(emb_hbm.at[uix], uft, sem)
    ci = pltpu.async_copy(emb_hbm.at[iix], ift, sem)
    cbu = pltpu.async_copy(bias_hbm.at[ruix], ub, sem)
    cbi = pltpu.async_copy(bias_hbm.at[riix], ib, sem)
    cu.wait()
    ci.wait()
    cbu.wait()
    cbi.wait()

    def dot_blk(blk, _):
        acc = ub[pl.ds(blk * L, L)] + ib[pl.ds(blk * L, L)]
        for d in range(D):
            u = uft[pl.ds(d * BPW + blk * L, L)]
            v = ift[pl.ds(d * BPW + blk * L, L)]
            acc = acc + u * v
        outv[pl.ds(blk * L, L)] = acc
        return 0

    lax.fori_loop(0, NBLK, dot_blk, 0)

    pltpu.sync_copy(outv, out_hbm.at[pl.ds(base, BPW)])


@jax.jit
def _gmf(x_batch, emb_table, bias_table):
    x = x_batch.astype(jnp.int32)
    x_flat = x.reshape(128, 128, 2).transpose(0, 2, 1).reshape(-1)
    emb_flat = (emb_table.reshape(RG, 128, 2, 8)
                .transpose(2, 0, 3, 1).reshape(-1))
    bias_flat = bias_table.reshape(-1)

    mesh = plsc.VectorSubcoreMesh(
        core_axis_name="c", subcore_axis_name="s",
        num_cores=NC, num_subcores=NS)
    return pl.kernel(
        _sc_body,
        out_type=jax.ShapeDtypeStruct((B,), jnp.float32),
        mesh=mesh,
        compiler_params=pltpu.CompilerParams(
            needs_layout_passes=False, use_tc_tiling_on_sc=False),
        scratch_types=[
            pltpu.VMEM((2 * BPW,), jnp.int32),
            pltpu.VMEM((D * BPW,), jnp.int32),
            pltpu.VMEM((D * BPW,), jnp.int32),
            pltpu.VMEM((BPW,), jnp.int32),
            pltpu.VMEM((BPW,), jnp.int32),
            pltpu.VMEM((D * BPW,), jnp.float32),
            pltpu.VMEM((D * BPW,), jnp.float32),
            pltpu.VMEM((BPW,), jnp.float32),
            pltpu.VMEM((BPW,), jnp.float32),
            pltpu.VMEM((BPW,), jnp.float32),
            pltpu.SemaphoreType.DMA,
        ],
    )(x_flat, emb_flat, bias_flat)


def kernel(x_batch, emb_table, bias_table):
    return _gmf(x_batch, emb_table, bias_table)


# speedup vs baseline: 9.1004x; 1.1922x over previous
import jax
import jax.numpy as jnp
from jax import lax
from jax.experimental import pallas as pl
from jax.experimental.pallas import tpu as pltpu
from jax.experimental.pallas import tpu_sc as plsc

N_USERS = 1000000
N_ITEMS = 1000000
D = 16
B = 16384

NC = 2
NS = 16
L = 16
NW = NC * NS
BPW = B // NW
NBLK = BPW // L

ROWS = N_USERS + N_ITEMS
RG = ROWS // 128
FG_STRIDE = RG * 1024

DOFF = [(d // 8) * FG_STRIDE + (d % 8) * 128 for d in range(D)]

CP = pltpu.CompilerParams(needs_layout_passes=False,
                          use_tc_tiling_on_sc=False)


def _mesh():
    return plsc.VectorSubcoreMesh(
        core_axis_name="c", subcore_axis_name="s",
        num_cores=NC, num_subcores=NS)


def _dot_body(x_hbm, emb_hbm, out_hbm, xv, uix, iix, uft, ift, outv, sem):
    wid = lax.axis_index("s") * NC + lax.axis_index("c")
    base = wid * BPW

    pltpu.sync_copy(x_hbm.at[pl.ds(base * 2, 2 * BPW)], xv)

    def build(blk, _):
        qoff = (blk // 8) * 256 + (blk % 8) * L
        ru = xv[pl.ds(qoff, L)]
        ri = xv[pl.ds(qoff + 128, L)] + N_USERS
        bu = ((ru >> 7) << 10) + (ru & 127)
        bi = ((ri >> 7) << 10) + (ri & 127)
        for d in range(D):
            uix[pl.ds(d * BPW + blk * L, L)] = bu + DOFF[d]
            iix[pl.ds(d * BPW + blk * L, L)] = bi + DOFF[d]
        return 0

    lax.fori_loop(0, NBLK, build, 0)

    cu = pltpu.async_copy(emb_hbm.at[uix], uft, sem)
    ci = pltpu.async_copy(emb_hbm.at[iix], ift, sem)
    cu.wait()
    ci.wait()

    def dot_blk(blk, _):
        acc = (uft[pl.ds(blk * L, L)] * ift[pl.ds(blk * L, L)])
        for d in range(1, D):
            u = uft[pl.ds(d * BPW + blk * L, L)]
            v = ift[pl.ds(d * BPW + blk * L, L)]
            acc = acc + u * v
        outv[pl.ds(blk * L, L)] = acc
        return 0

    lax.fori_loop(0, NBLK, dot_blk, 0)

    pltpu.sync_copy(outv, out_hbm.at[pl.ds(base, BPW)])


def _bias_body(x_hbm, bias_hbm, part_hbm, out_hbm,
               xv, ruix, riix, ub, ib, pv, sem):
    wid = lax.axis_index("s") * NC + lax.axis_index("c")
    base = wid * BPW

    pltpu.sync_copy(x_hbm.at[pl.ds(base * 2, 2 * BPW)], xv)
    cp = pltpu.async_copy(part_hbm.at[pl.ds(base, BPW)], pv, sem)

    def build(blk, _):
        qoff = (blk // 8) * 256 + (blk % 8) * L
        ruix[pl.ds(blk * L, L)] = xv[pl.ds(qoff, L)]
        riix[pl.ds(blk * L, L)] = xv[pl.ds(qoff + 128, L)] + N_USERS
        return 0

    lax.fori_loop(0, NBLK, build, 0)

    cu = pltpu.async_copy(bias_hbm.at[ruix], ub, sem)
    ci = pltpu.async_copy(bias_hbm.at[riix], ib, sem)
    cp.wait()
    cu.wait()
    ci.wait()

    def add_blk(blk, _):
        s = pl.ds(blk * L, L)
        pv[s] = pv[s] + ub[s] + ib[s]
        return 0

    lax.fori_loop(0, NBLK, add_blk, 0)

    pltpu.sync_copy(pv, out_hbm.at[pl.ds(base, BPW)])


@jax.jit
def _gmf(x_batch, emb_table, bias_table):
    x = x_batch.astype(jnp.int32)
    x_flat = x.reshape(128, 128, 2).transpose(0, 2, 1).reshape(-1)
    emb_flat = (emb_table.reshape(RG, 128, 2, 8)
                .transpose(2, 0, 3, 1).reshape(-1))
    bias_flat = jnp.sum(bias_table.T, axis=0)

    part = pl.kernel(
        _dot_body,
        out_type=jax.ShapeDtypeStruct((B,), jnp.float32),
        mesh=_mesh(),
        compiler_params=CP,
        scratch_types=[
            pltpu.VMEM((2 * BPW,), jnp.int32),
            pltpu.VMEM((D * BPW,), jnp.int32),
            pltpu.VMEM((D * BPW,), jnp.int32),
            pltpu.VMEM((D * BPW,), jnp.float32),
            pltpu.VMEM((D * BPW,), jnp.float32),
            pltpu.VMEM((BPW,), jnp.float32),
            pltpu.SemaphoreType.DMA,
        ],
    )(x_flat, emb_flat)

    out = pl.kernel(
        _bias_body,
        out_type=jax.ShapeDtypeStruct((B,), jnp.float32),
        mesh=_mesh(),
        compiler_params=CP,
        scratch_types=[
            pltpu.VMEM((2 * BPW,), jnp.int32),
            pltpu.VMEM((BPW,), jnp.int32),
            pltpu.VMEM((BPW,), jnp.int32),
            pltpu.VMEM((BPW,), jnp.float32),
            pltpu.VMEM((BPW,), jnp.float32),
            pltpu.VMEM((BPW,), jnp.float32),
            pltpu.SemaphoreType.DMA,
        ],
    )(x_flat, bias_flat, part)
    return out


def kernel(x_batch, emb_table, bias_table):
    return _gmf(x_batch, emb_table, bias_table)
